# baseline (device time: 20278 ns/iter reference)
import jax
import jax.numpy as jnp
from jax import lax
from jax.experimental import pallas as pl
from jax.experimental.pallas import tpu as pltpu

N_DEV = 32
N_PLANE = 8
N_Y = 4
N_Z = 4


def kernel(A, B):
    m, k = A.shape
    _, n = B.shape
    m_per = m // N_DEV

    def body(
        a_ref, b_ref, out_ref,
        partb_ref, recvx_ref, accx_ref, accxb_ref,
        recvy_ref, accy_ref, accyb_ref, recvz_ref,
        sendx_sem, recvx_sem, sendy_sems, recvy_sems, sendz_sems, recvz_sems,
    ):
        me = lax.axis_index("i")
        z = lax.div(me, N_PLANE)
        q = lax.rem(me, N_PLANE)
        y = lax.div(q, 2)
        my_x = lax.rem(lax.rem(q, 2) + lax.rem(y, 2), 2)

        x_partner = me + 1 - 2 * lax.rem(me, 2)

        def y_peer(yp):
            return z * N_PLANE + 2 * yp + lax.rem(my_x + yp, 2)

        def z_peer(zp):
            return zp * N_PLANE + q

        barrier_sem = pltpu.get_barrier_semaphore()
        peers = [x_partner]
        peers += [y_peer(lax.rem(y + dy, N_Y)) for dy in range(1, N_Y)]
        peers += [z_peer(lax.rem(z + dz, N_Z)) for dz in range(1, N_Z)]
        for peer in peers:
            pl.semaphore_signal(
                barrier_sem, inc=1, device_id=(peer,),
                device_id_type=pl.DeviceIdType.MESH,
            )

        part = jnp.dot(a_ref[...], b_ref[...], preferred_element_type=jnp.float32)
        for c in range(N_DEV):
            q_c, z_c = c % N_PLANE, c // N_PLANE
            y_c, x_c = q_c // 2, (q_c % 2) ^ ((q_c // 2) % 2)
            partb_ref[x_c, y_c, z_c] = part[
                c * m_per:(c + 1) * m_per, :
            ].astype(jnp.bfloat16)

        pl.semaphore_wait(barrier_sem, len(peers))

        rdma_x = pltpu.make_async_remote_copy(
            src_ref=partb_ref.at[1 - my_x],
            dst_ref=recvx_ref,
            send_sem=sendx_sem,
            recv_sem=recvx_sem,
            device_id=(x_partner,),
            device_id_type=pl.DeviceIdType.MESH,
        )
        rdma_x.start()
        rdma_x.wait()

        accx = partb_ref[my_x].astype(jnp.float32) + recvx_ref[...].astype(
            jnp.float32
        )
        accx_ref[...] = accx
        accxb_ref[...] = accx.astype(jnp.bfloat16)

        rdmas_y = []
        for dy in range(1, N_Y):
            yp = lax.rem(y + dy, N_Y)
            rdma = pltpu.make_async_remote_copy(
                src_ref=accxb_ref.at[yp],
                dst_ref=recvy_ref.at[dy],
                send_sem=sendy_sems.at[dy],
                recv_sem=recvy_sems.at[dy],
                device_id=(y_peer(yp),),
                device_id_type=pl.DeviceIdType.MESH,
            )
            rdma.start()
            rdmas_y.append(rdma)
        for rdma in rdmas_y:
            rdma.wait()

        accy = accx_ref[y] + jnp.sum(
            recvy_ref[1:].astype(jnp.float32), axis=0
        )
        accy_ref[...] = accy
        accyb_ref[...] = accy.astype(jnp.bfloat16)

        rdmas_z = []
        for dz in range(1, N_Z):
            zp = lax.rem(z + dz, N_Z)
            rdma = pltpu.make_async_remote_copy(
                src_ref=accyb_ref.at[zp],
                dst_ref=recvz_ref.at[dz],
                send_sem=sendz_sems.at[dz],
                recv_sem=recvz_sems.at[dz],
                device_id=(z_peer(zp),),
                device_id_type=pl.DeviceIdType.MESH,
            )
            rdma.start()
            rdmas_z.append(rdma)
        for rdma in rdmas_z:
            rdma.wait()

        out_ref[...] = accy_ref[z] + jnp.sum(
            recvz_ref[1:].astype(jnp.float32), axis=0
        )

    return pl.pallas_call(
        body,
        out_shape=jax.ShapeDtypeStruct((m_per, n), jnp.float32),
        in_specs=[
            pl.BlockSpec(memory_space=pltpu.VMEM),
            pl.BlockSpec(memory_space=pltpu.VMEM),
        ],
        out_specs=pl.BlockSpec(memory_space=pltpu.VMEM),
        scratch_shapes=[
            pltpu.VMEM((2, N_Y, N_Z, m_per, n), jnp.bfloat16),
            pltpu.VMEM((N_Y, N_Z, m_per, n), jnp.bfloat16),
            pltpu.VMEM((N_Y, N_Z, m_per, n), jnp.float32),
            pltpu.VMEM((N_Y, N_Z, m_per, n), jnp.bfloat16),
            pltpu.VMEM((N_Y, N_Z, m_per, n), jnp.bfloat16),
            pltpu.VMEM((N_Z, m_per, n), jnp.float32),
            pltpu.VMEM((N_Z, m_per, n), jnp.bfloat16),
            pltpu.VMEM((N_Z, m_per, n), jnp.bfloat16),
            pltpu.SemaphoreType.DMA,
            pltpu.SemaphoreType.DMA,
            pltpu.SemaphoreType.DMA((N_Y,)),
            pltpu.SemaphoreType.DMA((N_Y,)),
            pltpu.SemaphoreType.DMA((N_Z,)),
            pltpu.SemaphoreType.DMA((N_Z,)),
        ],
        compiler_params=pltpu.CompilerParams(collective_id=0),
    )(A, B)


# device time: 17783 ns/iter; 1.1403x vs baseline; 1.1403x over previous
import jax
import jax.numpy as jnp
from jax import lax
from jax.experimental import pallas as pl
from jax.experimental.pallas import tpu as pltpu

N_DEV = 32
N_PLANE = 8
N_Z = 4


def kernel(A, B):
    m, k = A.shape
    _, n = B.shape
    m_per = m // N_DEV

    def body(
        a_ref, b_ref, out_ref,
        part_ref, part2b_ref, recv1_ref, acc1_ref, acc2b_ref, recv2_ref,
        send_sems1, recv_sems1, send_sems2, recv_sems2,
    ):
        me = lax.axis_index("i")
        z = lax.div(me, N_PLANE)
        q = lax.rem(me, N_PLANE)

        barrier_sem = pltpu.get_barrier_semaphore()
        n_peers = 0
        for oq in range(1, N_PLANE):
            peer = z * N_PLANE + lax.rem(q + oq, N_PLANE)
            pl.semaphore_signal(
                barrier_sem, inc=1, device_id=(peer,),
                device_id_type=pl.DeviceIdType.MESH,
            )
            n_peers += 1
        for oz in range(1, N_Z):
            peer = lax.rem(z + oz, N_Z) * N_PLANE + q
            pl.semaphore_signal(
                barrier_sem, inc=1, device_id=(peer,),
                device_id_type=pl.DeviceIdType.MESH,
            )
            n_peers += 1

        part = jnp.dot(a_ref[...], b_ref[...], preferred_element_type=jnp.float32)
        part_ref[...] = part
        for c in range(N_DEV):
            part2b_ref[c % N_PLANE, c // N_PLANE] = part[
                c * m_per:(c + 1) * m_per, :
            ].astype(jnp.bfloat16)

        pl.semaphore_wait(barrier_sem, n_peers)

        rdmas1 = []
        for oq in range(1, N_PLANE):
            qp = lax.rem(q + oq, N_PLANE)
            target = z * N_PLANE + qp
            rdma = pltpu.make_async_remote_copy(
                src_ref=part2b_ref.at[qp],
                dst_ref=recv1_ref.at[oq],
                send_sem=send_sems1.at[oq],
                recv_sem=recv_sems1.at[oq],
                device_id=(target,),
                device_id_type=pl.DeviceIdType.MESH,
            )
            rdma.start()
            rdmas1.append(rdma)
        for rdma in rdmas1:
            rdma.wait_recv()

        for zp in range(N_Z):
            own = part_ref[pl.ds((zp * N_PLANE) * m_per + q * m_per, m_per), :]
            acc = own + jnp.sum(
                recv1_ref[1:, zp, :, :].astype(jnp.float32), axis=0
            )
            acc1_ref[zp] = acc
            acc2b_ref[zp] = acc.astype(jnp.bfloat16)

        rdmas2 = []
        for oz in range(1, N_Z):
            zp = lax.rem(z + oz, N_Z)
            target = zp * N_PLANE + q
            rdma = pltpu.make_async_remote_copy(
                src_ref=acc2b_ref.at[zp],
                dst_ref=recv2_ref.at[oz],
                send_sem=send_sems2.at[oz],
                recv_sem=recv_sems2.at[oz],
                device_id=(target,),
                device_id_type=pl.DeviceIdType.MESH,
            )
            rdma.start()
            rdmas2.append(rdma)
        for rdma in rdmas2:
            rdma.wait_recv()

        out_ref[...] = acc1_ref[z] + jnp.sum(
            recv2_ref[1:, :, :].astype(jnp.float32), axis=0
        )

        for rdma in rdmas1:
            rdma.wait_send()
        for rdma in rdmas2:
            rdma.wait_send()

    return pl.pallas_call(
        body,
        out_shape=jax.ShapeDtypeStruct((m_per, n), jnp.float32),
        in_specs=[
            pl.BlockSpec(memory_space=pltpu.VMEM),
            pl.BlockSpec(memory_space=pltpu.VMEM),
        ],
        out_specs=pl.BlockSpec(memory_space=pltpu.VMEM),
        scratch_shapes=[
            pltpu.VMEM((m, n), jnp.float32),
            pltpu.VMEM((N_PLANE, N_Z, m_per, n), jnp.bfloat16),
            pltpu.VMEM((N_PLANE, N_Z, m_per, n), jnp.bfloat16),
            pltpu.VMEM((N_Z, m_per, n), jnp.float32),
            pltpu.VMEM((N_Z, m_per, n), jnp.bfloat16),
            pltpu.VMEM((N_Z, m_per, n), jnp.bfloat16),
            pltpu.SemaphoreType.DMA((N_PLANE,)),
            pltpu.SemaphoreType.DMA((N_PLANE,)),
            pltpu.SemaphoreType.DMA((N_Z,)),
            pltpu.SemaphoreType.DMA((N_Z,)),
        ],
        compiler_params=pltpu.CompilerParams(collective_id=0),
    )(A, B)
